# baseline (speedup 1.0000x reference)
import functools

import jax
import jax.numpy as jnp
import numpy as np
from jax import lax
from jax.experimental import pallas as pl
from jax.experimental.pallas import tpu as pltpu
from jax.experimental.pallas import tpu_sc as plsc

B, S, H = 16, 2048, 768
N = B * S
HP = H // 2
NC, NS = 2, 16
NW = NC * NS
CHUNK = 32

NSPLIT = 8
ROWS = N // NSPLIT
PW = ROWS // NW
NCH = PW // CHUNK

BR = 512
SPB = S // BR
CBLK = ROWS // BR

L = 16
NG = H // (2 * L)

_PERM_LO = np.empty(HP, np.int64)
_PERM_HI = np.empty(HP, np.int64)
for _g in range(NG):
    for _i in range(L):
        _PERM_LO[_g * L + _i] = _g * 2 * L + _i
        _PERM_HI[_g * L + _i] = _g * 2 * L + L + _i


def _sc_gather_pack(ids, table):
    mesh = plsc.VectorSubcoreMesh(core_axis_name="c", subcore_axis_name="s")

    @functools.partial(
        pl.kernel,
        mesh=mesh,
        compiler_params=pltpu.CompilerParams(needs_layout_passes=False),
        out_type=jax.ShapeDtypeStruct((ROWS, HP), jnp.int32),
        scratch_types=[
            pltpu.VMEM((PW,), jnp.int32),
            pltpu.VMEM((CHUNK, H), jnp.float32),
            pltpu.VMEM((CHUNK, H), jnp.float32),
            pltpu.VMEM((CHUNK, HP), jnp.int32),
            pltpu.VMEM((CHUNK, HP), jnp.int32),
            pltpu.SemaphoreType.DMA,
            pltpu.SemaphoreType.DMA,
            pltpu.SemaphoreType.DMA,
            pltpu.SemaphoreType.DMA,
        ],
    )
    def k(ids_hbm, table_hbm, out_hbm, idx_v, buf0, buf1, pb0, pb1,
          sg0, sg1, so0, so1):
        wid = lax.axis_index("s") * NC + lax.axis_index("c")
        base = wid * PW
        pltpu.sync_copy(ids_hbm.at[pl.ds(base, PW)], idx_v)
        bufs = (buf0, buf1)
        pbufs = (pb0, pb1)
        gsems = (sg0, sg1)
        osems = (so0, so1)

        def start_gather(c):
            return pltpu.async_copy(
                table_hbm.at[idx_v.at[pl.ds(c * CHUNK, CHUNK)]],
                bufs[c % 2], gsems[c % 2])

        def convert(buf, pbuf):
            def row(r, _):
                for g in range(NG):
                    va = buf[r, pl.ds(g * 2 * L, L)]
                    vb = buf[r, pl.ds(g * 2 * L + L, L)]
                    packed = plsc.pack(va, vb,
                                       format=plsc.PackFormat.INTERLEAVED)
                    pbuf[r, pl.ds(g * L, L)] = plsc.bitcast(
                        packed, jnp.int32)
                return _
            lax.fori_loop(0, CHUNK, row, 0)

        ghandles = [None] * NCH
        ohandles = [None] * NCH
        ghandles[0] = start_gather(0)
        for c in range(NCH):
            ghandles[c].wait()
            if c + 1 < NCH:
                ghandles[c + 1] = start_gather(c + 1)
            if c >= 2:
                ohandles[c - 2].wait()
            convert(bufs[c % 2], pbufs[c % 2])
            ohandles[c] = pltpu.async_copy(
                pbufs[c % 2],
                out_hbm.at[pl.ds(base + c * CHUNK, CHUNK)],
                osems[c % 2])
        ohandles[NCH - 2].wait()
        ohandles[NCH - 1].wait()

    return k(ids, table)


def _bias_body(t_i, d_i, m_i, h_i, l_i, r_i, w_i,
               t_b, d_b, m_b, h_b, l_b, r_b, w_b, wr_ref, pb_ref, o_ref):
    r = jnp.concatenate(
        [t_b[0], d_b[0], m_b[0], h_b[0], l_b[0], r_b[0], w_b[0]],
        axis=-1)
    o_ref[...] = lax.dot_general(
        r, wr_ref[...], (((1,), (1,)), ((), ())),
        preferred_element_type=jnp.float32)[None] + pb_ref[...]


def _bias16(idxs, tables, w_rest, proj_b2d):
    q = H // 4
    in_specs = []
    for k in range(7):
        width = (H // 2) if k == 4 else q
        in_specs.append(pl.BlockSpec(
            (1, 1, width), lambda b, *s, _k=k: (s[_k][b], 0, 0)))
    in_specs.append(pl.BlockSpec((H, 2 * H), lambda b, *s: (0, 0)))
    in_specs.append(pl.BlockSpec((1, H), lambda b, *s: (0, 0)))
    grid_spec = pltpu.PrefetchScalarGridSpec(
        num_scalar_prefetch=7,
        grid=(B,),
        in_specs=in_specs,
        out_specs=pl.BlockSpec((1, 1, H), lambda b, *s: (b, 0, 0)),
    )
    tables3d = tuple(t[:, None, :] for t in tables)
    return pl.pallas_call(
        _bias_body,
        grid_spec=grid_spec,
        out_shape=jax.ShapeDtypeStruct((B, 1, H), jnp.float32),
    )(*idxs, *tables3d, w_rest, proj_b2d)


def _proj_body(prev_ref, x_ref, wlo_ref, whi_ref, bias_ref, nw_ref, o_ref):
    xi = x_ref[...]
    xlo = lax.bitcast_convert_type(xi << 16, jnp.float32)
    xhi = lax.bitcast_convert_type(
        xi & jnp.int32(-65536), jnp.float32)
    y = lax.dot_general(
        xlo, wlo_ref[...], (((1,), (1,)), ((), ())),
        preferred_element_type=jnp.float32)
    y = y + lax.dot_general(
        xhi, whi_ref[...], (((1,), (1,)), ((), ())),
        preferred_element_type=jnp.float32)
    y = y + bias_ref[0]
    ms = jnp.mean(y * y, axis=-1, keepdims=True)
    o_ref[...] = y * lax.rsqrt(ms + 1e-6) * nw_ref[...]


def _proj_body_first(x_ref, wlo_ref, whi_ref, bias_ref, nw_ref, o_ref):
    _proj_body(None, x_ref, wlo_ref, whi_ref, bias_ref, nw_ref, o_ref)


def _project_split(k, prev, x, wlo, whi, bias, norm_w2d):
    base = k * CBLK
    data_specs = [
        pl.BlockSpec((BR, HP), lambda i: (i, 0)),
        pl.BlockSpec((H, HP), lambda i: (0, 0)),
        pl.BlockSpec((H, HP), lambda i: (0, 0)),
        pl.BlockSpec((1, 1, H), lambda i: ((base + i) // SPB, 0, 0)),
        pl.BlockSpec((1, H), lambda i: (0, 0)),
    ]
    out_spec = pl.BlockSpec((BR, H), lambda i: (base + i, 0))
    out_shape = jax.ShapeDtypeStruct((N, H), jnp.float32)
    if prev is None:
        return pl.pallas_call(
            _proj_body_first,
            grid=(CBLK,),
            in_specs=data_specs,
            out_specs=out_spec,
            out_shape=out_shape,
        )(x, wlo, whi, bias, norm_w2d)
    return pl.pallas_call(
        _proj_body,
        grid=(CBLK,),
        in_specs=[pl.BlockSpec((BR, H), lambda i: (0, 0))] + data_specs,
        out_specs=out_spec,
        out_shape=out_shape,
        input_output_aliases={0: 0},
    )(prev, x, wlo, whi, bias, norm_w2d)


def kernel(input_ids, time_slots, day_of_week, month, is_holiday,
           location_ids, road_types, weather_states, word_table, time_table,
           dow_table, month_table, holiday_table, loc_table, road_table,
           weather_table, proj_w, proj_b, norm_w):
    ids = input_ids.reshape(-1).astype(jnp.int32)
    w_rest = proj_w[:, H:]
    w_word = proj_w[:, :H]
    wlo = w_word[:, _PERM_LO]
    whi = w_word[:, _PERM_HI]

    xs = [_sc_gather_pack(lax.slice(ids, (k * ROWS,), ((k + 1) * ROWS,)),
                          word_table)
          for k in range(NSPLIT)]

    idxs = tuple(a.reshape(-1).astype(jnp.int32) for a in
                 (time_slots, day_of_week, month, is_holiday,
                  location_ids, road_types, weather_states))
    tables = (time_table, dow_table, month_table, holiday_table,
              loc_table, road_table, weather_table)
    bias = _bias16(idxs, tables, w_rest, proj_b.reshape(1, H))

    norm_w2d = norm_w.reshape(1, H)
    out = None
    for k in range(NSPLIT):
        out = _project_split(k, out, xs[k], wlo, whi, bias, norm_w2d)
    return out.reshape(B, S, H)


# speedup vs baseline: 1.1075x; 1.1075x over previous
import functools

import jax
import jax.numpy as jnp
import numpy as np
from jax import lax
from jax.experimental import pallas as pl
from jax.experimental.pallas import tpu as pltpu
from jax.experimental.pallas import tpu_sc as plsc

B, S, H = 16, 2048, 768
N = B * S
HP = H // 2
NC, NS = 2, 16
NW = NC * NS
CHUNK = 32

NSPLIT = 4
ROWS = N // NSPLIT
PW = ROWS // NW
NCH = PW // CHUNK

BR = 1024
SPB = S // BR
CBLK = ROWS // BR

L = 16
NG = H // (2 * L)

_PERM_LO = np.empty(HP, np.int64)
_PERM_HI = np.empty(HP, np.int64)
for _g in range(NG):
    for _i in range(L):
        _PERM_LO[_g * L + _i] = _g * 2 * L + _i
        _PERM_HI[_g * L + _i] = _g * 2 * L + L + _i


def _sc_gather_pack(ids, table):
    mesh = plsc.VectorSubcoreMesh(core_axis_name="c", subcore_axis_name="s")

    @functools.partial(
        pl.kernel,
        mesh=mesh,
        compiler_params=pltpu.CompilerParams(needs_layout_passes=False),
        out_type=jax.ShapeDtypeStruct((ROWS, HP), jnp.int32),
        scratch_types=[
            pltpu.VMEM((PW,), jnp.int32),
            pltpu.VMEM((CHUNK, H), jnp.float32),
            pltpu.VMEM((CHUNK, H), jnp.float32),
            pltpu.VMEM((CHUNK, HP), jnp.int32),
            pltpu.VMEM((CHUNK, HP), jnp.int32),
            pltpu.SemaphoreType.DMA,
            pltpu.SemaphoreType.DMA,
            pltpu.SemaphoreType.DMA,
            pltpu.SemaphoreType.DMA,
        ],
    )
    def k(ids_hbm, table_hbm, out_hbm, idx_v, buf0, buf1, pb0, pb1,
          sg0, sg1, so0, so1):
        wid = lax.axis_index("s") * NC + lax.axis_index("c")
        base = wid * PW
        pltpu.sync_copy(ids_hbm.at[pl.ds(base, PW)], idx_v)
        bufs = (buf0, buf1)
        pbufs = (pb0, pb1)
        gsems = (sg0, sg1)
        osems = (so0, so1)

        def start_gather(c):
            return pltpu.async_copy(
                table_hbm.at[idx_v.at[pl.ds(c * CHUNK, CHUNK)]],
                bufs[c % 2], gsems[c % 2])

        def convert(buf, pbuf):
            def row(r, _):
                for g in range(NG):
                    va = buf[r, pl.ds(g * 2 * L, L)]
                    vb = buf[r, pl.ds(g * 2 * L + L, L)]
                    packed = plsc.pack(va, vb,
                                       format=plsc.PackFormat.INTERLEAVED)
                    pbuf[r, pl.ds(g * L, L)] = plsc.bitcast(
                        packed, jnp.int32)
                return _
            lax.fori_loop(0, CHUNK, row, 0)

        ghandles = [None] * NCH
        ohandles = [None] * NCH
        ghandles[0] = start_gather(0)
        for c in range(NCH):
            ghandles[c].wait()
            if c + 1 < NCH:
                ghandles[c + 1] = start_gather(c + 1)
            if c >= 2:
                ohandles[c - 2].wait()
            convert(bufs[c % 2], pbufs[c % 2])
            ohandles[c] = pltpu.async_copy(
                pbufs[c % 2],
                out_hbm.at[pl.ds(base + c * CHUNK, CHUNK)],
                osems[c % 2])
        ohandles[NCH - 2].wait()
        ohandles[NCH - 1].wait()

    return k(ids, table)


def _bias_body(t_i, d_i, m_i, h_i, l_i, r_i, w_i,
               t_b, d_b, m_b, h_b, l_b, r_b, w_b, wr_ref, pb_ref, o_ref):
    r = jnp.concatenate(
        [t_b[0], d_b[0], m_b[0], h_b[0], l_b[0], r_b[0], w_b[0]],
        axis=-1)
    o_ref[...] = lax.dot_general(
        r, wr_ref[...], (((1,), (1,)), ((), ())),
        preferred_element_type=jnp.float32)[None] + pb_ref[...]


def _bias16(idxs, tables, w_rest, proj_b2d):
    q = H // 4
    in_specs = []
    for k in range(7):
        width = (H // 2) if k == 4 else q
        in_specs.append(pl.BlockSpec(
            (1, 1, width), lambda b, *s, _k=k: (s[_k][b], 0, 0)))
    in_specs.append(pl.BlockSpec((H, 2 * H), lambda b, *s: (0, 0)))
    in_specs.append(pl.BlockSpec((1, H), lambda b, *s: (0, 0)))
    grid_spec = pltpu.PrefetchScalarGridSpec(
        num_scalar_prefetch=7,
        grid=(B,),
        in_specs=in_specs,
        out_specs=pl.BlockSpec((1, 1, H), lambda b, *s: (b, 0, 0)),
    )
    tables3d = tuple(t[:, None, :] for t in tables)
    return pl.pallas_call(
        _bias_body,
        grid_spec=grid_spec,
        out_shape=jax.ShapeDtypeStruct((B, 1, H), jnp.float32),
    )(*idxs, *tables3d, w_rest, proj_b2d)


def _proj_body(prev_ref, x_ref, wlo_ref, whi_ref, bias_ref, nw_ref, o_ref):
    xi = x_ref[...]
    xlo = lax.bitcast_convert_type(xi << 16, jnp.float32)
    xhi = lax.bitcast_convert_type(
        xi & jnp.int32(-65536), jnp.float32)
    y = lax.dot_general(
        xlo, wlo_ref[...], (((1,), (1,)), ((), ())),
        preferred_element_type=jnp.float32)
    y = y + lax.dot_general(
        xhi, whi_ref[...], (((1,), (1,)), ((), ())),
        preferred_element_type=jnp.float32)
    y = y + bias_ref[0]
    ms = jnp.mean(y * y, axis=-1, keepdims=True)
    o_ref[...] = y * lax.rsqrt(ms + 1e-6) * nw_ref[...]


def _proj_body_first(x_ref, wlo_ref, whi_ref, bias_ref, nw_ref, o_ref):
    _proj_body(None, x_ref, wlo_ref, whi_ref, bias_ref, nw_ref, o_ref)


def _project_split(k, prev, x, wlo, whi, bias, norm_w2d):
    base = k * CBLK
    data_specs = [
        pl.BlockSpec((BR, HP), lambda i: (i, 0)),
        pl.BlockSpec((H, HP), lambda i: (0, 0)),
        pl.BlockSpec((H, HP), lambda i: (0, 0)),
        pl.BlockSpec((1, 1, H), lambda i: ((base + i) // SPB, 0, 0)),
        pl.BlockSpec((1, H), lambda i: (0, 0)),
    ]
    out_spec = pl.BlockSpec((BR, H), lambda i: (base + i, 0))
    out_shape = jax.ShapeDtypeStruct((N, H), jnp.float32)
    if prev is None:
        return pl.pallas_call(
            _proj_body_first,
            grid=(CBLK,),
            in_specs=data_specs,
            out_specs=out_spec,
            out_shape=out_shape,
        )(x, wlo, whi, bias, norm_w2d)
    return pl.pallas_call(
        _proj_body,
        grid=(CBLK,),
        in_specs=[pl.BlockSpec((BR, H), lambda i: (0, 0))] + data_specs,
        out_specs=out_spec,
        out_shape=out_shape,
        input_output_aliases={0: 0},
    )(prev, x, wlo, whi, bias, norm_w2d)


def kernel(input_ids, time_slots, day_of_week, month, is_holiday,
           location_ids, road_types, weather_states, word_table, time_table,
           dow_table, month_table, holiday_table, loc_table, road_table,
           weather_table, proj_w, proj_b, norm_w):
    ids = input_ids.reshape(-1).astype(jnp.int32)
    w_rest = proj_w[:, H:]
    w_word = proj_w[:, :H]
    wlo = w_word[:, _PERM_LO]
    whi = w_word[:, _PERM_HI]

    xs = [_sc_gather_pack(lax.slice(ids, (k * ROWS,), ((k + 1) * ROWS,)),
                          word_table)
          for k in range(NSPLIT)]

    idxs = tuple(a.reshape(-1).astype(jnp.int32) for a in
                 (time_slots, day_of_week, month, is_holiday,
                  location_ids, road_types, weather_states))
    tables = (time_table, dow_table, month_table, holiday_table,
              loc_table, road_table, weather_table)
    bias = _bias16(idxs, tables, w_rest, proj_b.reshape(1, H))

    norm_w2d = norm_w.reshape(1, H)
    out = None
    for k in range(NSPLIT):
        out = _project_split(k, out, xs[k], wlo, whi, bias, norm_w2d)
    return out.reshape(B, S, H)


# speedup vs baseline: 1.1945x; 1.0785x over previous
import functools

import jax
import jax.numpy as jnp
from jax import lax
from jax.experimental import pallas as pl
from jax.experimental.pallas import tpu as pltpu
from jax.experimental.pallas import tpu_sc as plsc

B, S, H = 16, 2048, 768
N = B * S
NC, NS = 2, 16
NW = NC * NS
CHUNK = 64

NSPLIT = 4
ROWS = N // NSPLIT
PW = ROWS // NW
NCH = PW // CHUNK

BR = 1024
SPB = S // BR
CBLK = ROWS // BR


def _sc_gather(ids, table):
    mesh = plsc.VectorSubcoreMesh(core_axis_name="c", subcore_axis_name="s")

    @functools.partial(
        pl.kernel,
        mesh=mesh,
        out_type=jax.ShapeDtypeStruct((ROWS, H), jnp.float32),
        scratch_types=[
            pltpu.VMEM((PW,), jnp.int32),
            pltpu.VMEM((CHUNK, H), jnp.float32),
            pltpu.VMEM((CHUNK, H), jnp.float32),
            pltpu.SemaphoreType.DMA,
            pltpu.SemaphoreType.DMA,
        ],
    )
    def k(ids_hbm, table_hbm, out_hbm, idx_v, buf0, buf1, sem0, sem1):
        wid = lax.axis_index("s") * NC + lax.axis_index("c")
        base = wid * PW
        pltpu.sync_copy(ids_hbm.at[pl.ds(base, PW)], idx_v)
        bufs = (buf0, buf1)
        sems = (sem0, sem1)

        def start(c):
            return pltpu.async_copy(
                table_hbm.at[idx_v.at[pl.ds(c * CHUNK, CHUNK)]],
                bufs[c % 2], sems[c % 2])

        handles = [None] * NCH
        handles[0] = start(0)
        for c in range(NCH):
            handles[c].wait()
            if c + 1 < NCH:
                handles[c + 1] = start(c + 1)
            pltpu.sync_copy(bufs[c % 2],
                            out_hbm.at[pl.ds(base + c * CHUNK, CHUNK)])

    return k(ids, table)


def _bias_body(t_i, d_i, m_i, h_i, l_i, r_i, w_i,
               t_b, d_b, m_b, h_b, l_b, r_b, w_b, wr_ref, pb_ref, o_ref):
    r = jnp.concatenate(
        [t_b[0], d_b[0], m_b[0], h_b[0], l_b[0], r_b[0], w_b[0]],
        axis=-1)
    o_ref[...] = lax.dot_general(
        r, wr_ref[...], (((1,), (1,)), ((), ())),
        preferred_element_type=jnp.float32)[None] + pb_ref[...]


def _bias16(idxs, tables, w_rest, proj_b2d):
    q = H // 4
    in_specs = []
    for k in range(7):
        width = (H // 2) if k == 4 else q
        in_specs.append(pl.BlockSpec(
            (1, 1, width), lambda b, *s, _k=k: (s[_k][b], 0, 0)))
    in_specs.append(pl.BlockSpec((H, 2 * H), lambda b, *s: (0, 0)))
    in_specs.append(pl.BlockSpec((1, H), lambda b, *s: (0, 0)))
    grid_spec = pltpu.PrefetchScalarGridSpec(
        num_scalar_prefetch=7,
        grid=(B,),
        in_specs=in_specs,
        out_specs=pl.BlockSpec((1, 1, H), lambda b, *s: (b, 0, 0)),
    )
    tables3d = tuple(t[:, None, :] for t in tables)
    return pl.pallas_call(
        _bias_body,
        grid_spec=grid_spec,
        out_shape=jax.ShapeDtypeStruct((B, 1, H), jnp.float32),
    )(*idxs, *tables3d, w_rest, proj_b2d)


def _proj_body(prev_ref, x_ref, w_ref, bias_ref, nw_ref, o_ref):
    y = lax.dot_general(
        x_ref[...], w_ref[...], (((1,), (1,)), ((), ())),
        preferred_element_type=jnp.float32)
    y = y + bias_ref[0]
    ms = jnp.mean(y * y, axis=-1, keepdims=True)
    o_ref[...] = y * lax.rsqrt(ms + 1e-6) * nw_ref[...]


def _proj_body_first(x_ref, w_ref, bias_ref, nw_ref, o_ref):
    _proj_body(None, x_ref, w_ref, bias_ref, nw_ref, o_ref)


def _project_split(k, prev, x, w_word, bias, norm_w2d):
    base = k * CBLK
    data_specs = [
        pl.BlockSpec((BR, H), lambda i: (i, 0)),
        pl.BlockSpec((H, H), lambda i: (0, 0)),
        pl.BlockSpec((1, 1, H), lambda i: ((base + i) // SPB, 0, 0)),
        pl.BlockSpec((1, H), lambda i: (0, 0)),
    ]
    out_spec = pl.BlockSpec((BR, H), lambda i: (base + i, 0))
    out_shape = jax.ShapeDtypeStruct((N, H), jnp.float32)
    if prev is None:
        return pl.pallas_call(
            _proj_body_first,
            grid=(CBLK,),
            in_specs=data_specs,
            out_specs=out_spec,
            out_shape=out_shape,
        )(x, w_word, bias, norm_w2d)
    return pl.pallas_call(
        _proj_body,
        grid=(CBLK,),
        in_specs=[pl.BlockSpec((BR, H), lambda i: (0, 0))] + data_specs,
        out_specs=out_spec,
        out_shape=out_shape,
        input_output_aliases={0: 0},
    )(prev, x, w_word, bias, norm_w2d)


def kernel(input_ids, time_slots, day_of_week, month, is_holiday,
           location_ids, road_types, weather_states, word_table, time_table,
           dow_table, month_table, holiday_table, loc_table, road_table,
           weather_table, proj_w, proj_b, norm_w):
    ids = input_ids.reshape(-1).astype(jnp.int32)
    w_word = proj_w[:, :H]
    w_rest = proj_w[:, H:]

    xs = [_sc_gather(lax.slice(ids, (k * ROWS,), ((k + 1) * ROWS,)),
                     word_table)
          for k in range(NSPLIT)]

    idxs = tuple(a.reshape(-1).astype(jnp.int32) for a in
                 (time_slots, day_of_week, month, is_holiday,
                  location_ids, road_types, weather_states))
    tables = (time_table, dow_table, month_table, holiday_table,
              loc_table, road_table, weather_table)
    bias = _bias16(idxs, tables, w_rest, proj_b.reshape(1, H))

    norm_w2d = norm_w.reshape(1, H)
    out = None
    for k in range(NSPLIT):
        out = _project_split(k, out, xs[k], w_word, bias, norm_w2d)
    return out.reshape(B, S, H)


# speedup vs baseline: 1.2035x; 1.0075x over previous
import functools

import jax
import jax.numpy as jnp
from jax import lax
from jax.experimental import pallas as pl
from jax.experimental.pallas import tpu as pltpu
from jax.experimental.pallas import tpu_sc as plsc

B, S, H = 16, 2048, 768
N = B * S
NC, NS = 2, 16
NW = NC * NS
CHUNK = 64

NSPLIT = 4
ROWS = N // NSPLIT
PW = ROWS // NW
NCH = PW // CHUNK

BR = 2048
SPB = S // BR
CBLK = ROWS // BR


def _sc_gather(ids, table):
    mesh = plsc.VectorSubcoreMesh(core_axis_name="c", subcore_axis_name="s")

    @functools.partial(
        pl.kernel,
        mesh=mesh,
        out_type=jax.ShapeDtypeStruct((ROWS, H), jnp.float32),
        scratch_types=[
            pltpu.VMEM((PW,), jnp.int32),
            pltpu.VMEM((CHUNK, H), jnp.float32),
            pltpu.VMEM((CHUNK, H), jnp.float32),
            pltpu.SemaphoreType.DMA,
            pltpu.SemaphoreType.DMA,
        ],
    )
    def k(ids_hbm, table_hbm, out_hbm, idx_v, buf0, buf1, sem0, sem1):
        wid = lax.axis_index("s") * NC + lax.axis_index("c")
        base = wid * PW
        pltpu.sync_copy(ids_hbm.at[pl.ds(base, PW)], idx_v)
        bufs = (buf0, buf1)
        sems = (sem0, sem1)

        def start(c):
            return pltpu.async_copy(
                table_hbm.at[idx_v.at[pl.ds(c * CHUNK, CHUNK)]],
                bufs[c % 2], sems[c % 2])

        handles = [None] * NCH
        handles[0] = start(0)
        for c in range(NCH):
            handles[c].wait()
            if c + 1 < NCH:
                handles[c + 1] = start(c + 1)
            pltpu.sync_copy(bufs[c % 2],
                            out_hbm.at[pl.ds(base + c * CHUNK, CHUNK)])

    return k(ids, table)


def _bias_body(t_i, d_i, m_i, h_i, l_i, r_i, w_i,
               t_b, d_b, m_b, h_b, l_b, r_b, w_b, wr_ref, pb_ref, o_ref):
    r = jnp.concatenate(
        [t_b[0], d_b[0], m_b[0], h_b[0], l_b[0], r_b[0], w_b[0]],
        axis=-1)
    o_ref[...] = lax.dot_general(
        r, wr_ref[...], (((1,), (1,)), ((), ())),
        preferred_element_type=jnp.float32)[None] + pb_ref[...]


def _bias16(idxs, tables, w_rest, proj_b2d):
    q = H // 4
    in_specs = []
    for k in range(7):
        width = (H // 2) if k == 4 else q
        in_specs.append(pl.BlockSpec(
            (1, 1, width), lambda b, *s, _k=k: (s[_k][b], 0, 0)))
    in_specs.append(pl.BlockSpec((H, 2 * H), lambda b, *s: (0, 0)))
    in_specs.append(pl.BlockSpec((1, H), lambda b, *s: (0, 0)))
    grid_spec = pltpu.PrefetchScalarGridSpec(
        num_scalar_prefetch=7,
        grid=(B,),
        in_specs=in_specs,
        out_specs=pl.BlockSpec((1, 1, H), lambda b, *s: (b, 0, 0)),
    )
    tables3d = tuple(t[:, None, :] for t in tables)
    return pl.pallas_call(
        _bias_body,
        grid_spec=grid_spec,
        out_shape=jax.ShapeDtypeStruct((B, 1, H), jnp.float32),
    )(*idxs, *tables3d, w_rest, proj_b2d)


def _proj_body(prev_ref, x_ref, w_ref, bias_ref, nw_ref, o_ref):
    y = lax.dot_general(
        x_ref[...], w_ref[...], (((1,), (1,)), ((), ())),
        preferred_element_type=jnp.float32)
    y = y + bias_ref[0]
    ms = jnp.mean(y * y, axis=-1, keepdims=True)
    o_ref[...] = y * lax.rsqrt(ms + 1e-6) * nw_ref[...]


def _proj_body_first(x_ref, w_ref, bias_ref, nw_ref, o_ref):
    _proj_body(None, x_ref, w_ref, bias_ref, nw_ref, o_ref)


def _project_split(k, prev, x, w_word, bias, norm_w2d):
    base = k * CBLK
    data_specs = [
        pl.BlockSpec((BR, H), lambda i: (i, 0)),
        pl.BlockSpec((H, H), lambda i: (0, 0)),
        pl.BlockSpec((1, 1, H), lambda i: ((base + i) // SPB, 0, 0)),
        pl.BlockSpec((1, H), lambda i: (0, 0)),
    ]
    out_spec = pl.BlockSpec((BR, H), lambda i: (base + i, 0))
    out_shape = jax.ShapeDtypeStruct((N, H), jnp.float32)
    if prev is None:
        return pl.pallas_call(
            _proj_body_first,
            grid=(CBLK,),
            in_specs=data_specs,
            out_specs=out_spec,
            out_shape=out_shape,
        )(x, w_word, bias, norm_w2d)
    return pl.pallas_call(
        _proj_body,
        grid=(CBLK,),
        in_specs=[pl.BlockSpec((BR, H), lambda i: (0, 0))] + data_specs,
        out_specs=out_spec,
        out_shape=out_shape,
        input_output_aliases={0: 0},
    )(prev, x, w_word, bias, norm_w2d)


def kernel(input_ids, time_slots, day_of_week, month, is_holiday,
           location_ids, road_types, weather_states, word_table, time_table,
           dow_table, month_table, holiday_table, loc_table, road_table,
           weather_table, proj_w, proj_b, norm_w):
    ids = input_ids.reshape(-1).astype(jnp.int32)
    w_word = proj_w[:, :H]
    w_rest = proj_w[:, H:]

    xs = [_sc_gather(lax.slice(ids, (k * ROWS,), ((k + 1) * ROWS,)),
                     word_table)
          for k in range(NSPLIT)]

    idxs = tuple(a.reshape(-1).astype(jnp.int32) for a in
                 (time_slots, day_of_week, month, is_holiday,
                  location_ids, road_types, weather_states))
    tables = (time_table, dow_table, month_table, holiday_table,
              loc_table, road_table, weather_table)
    bias = _bias16(idxs, tables, w_rest, proj_b.reshape(1, H))

    norm_w2d = norm_w.reshape(1, H)
    out = None
    for k in range(NSPLIT):
        out = _project_split(k, out, xs[k], w_word, bias, norm_w2d)
    return out.reshape(B, S, H)
